# baseline (speedup 1.0000x reference)
import functools

import jax
import jax.numpy as jnp
from jax import lax
from jax.experimental import pallas as pl
from jax.experimental.pallas import tpu as pltpu
from jax.experimental.pallas import tpu_sc as plsc

N = 10000
N_PAD = 10240
D = 256
H = 128
E = 160000
N_STEPS = 2
NS = 16
EB = 80
R = 1000


def _wh_body(h_ref, wt_ref, b_ref, lo_ref, hi_ref):
    wh = jnp.dot(h_ref[...], wt_ref[0], preferred_element_type=jnp.float32)
    wh = wh + b_ref[0]
    lo_ref[...] = wh[:, :H]
    hi_ref[...] = wh[:, H:]


def _wh_call(h, wT, b, K):
    nb = N // R
    return pl.pallas_call(
        _wh_body,
        grid=(K, nb),
        in_specs=[
            pl.BlockSpec((R, D), lambda k, i: (i, 0)),
            pl.BlockSpec((1, D, D), lambda k, i: (k, 0, 0)),
            pl.BlockSpec((1, 1, D), lambda k, i: (k, 0, 0)),
        ],
        out_specs=[
            pl.BlockSpec((R, H), lambda k, i: (k * nb + i, 0)),
            pl.BlockSpec((R, H), lambda k, i: (k * nb + i, 0)),
        ],
        out_shape=[
            jax.ShapeDtypeStruct((K * N, H), jnp.float32),
            jax.ShapeDtypeStruct((K * N, H), jnp.float32),
        ],
    )(h, wT, b)


def _gh_body(h_ref, whhT_ref, bhh_ref, out_ref):
    out_ref[...] = (jnp.dot(h_ref[...], whhT_ref[...],
                            preferred_element_type=jnp.float32)
                    + bhh_ref[...])


def _gh_call(h, whhT, bhh):
    nb = N // R
    return pl.pallas_call(
        _gh_body,
        grid=(nb,),
        in_specs=[
            pl.BlockSpec((R, D), lambda i: (i, 0)),
            pl.BlockSpec((D, 3 * D), lambda i: (0, 0)),
            pl.BlockSpec((1, 3 * D), lambda i: (0, 0)),
        ],
        out_specs=pl.BlockSpec((R, 3 * D), lambda i: (i, 0)),
        out_shape=jax.ShapeDtypeStruct((N, 3 * D), jnp.float32),
    )(h, whhT, bhh)


def _gru_body(alo_ref, ahi_ref, h_ref, wihT_ref, bih_ref, gh_ref, out_ref):
    gi = jnp.dot(alo_ref[...], wihT_ref[:H], preferred_element_type=jnp.float32)
    gi = gi + jnp.dot(ahi_ref[...], wihT_ref[H:],
                      preferred_element_type=jnp.float32)
    gi = gi + bih_ref[...]
    h = h_ref[...]
    gh = gh_ref[...]
    r = jax.nn.sigmoid(gi[:, :D] + gh[:, :D])
    z = jax.nn.sigmoid(gi[:, D:2 * D] + gh[:, D:2 * D])
    n = jnp.tanh(gi[:, 2 * D:] + r * gh[:, 2 * D:])
    out_ref[...] = (1.0 - z) * n + z * h


def _gru_call(alo, ahi, h, wihT, bih, gh):
    nb = N // R
    return pl.pallas_call(
        _gru_body,
        grid=(nb,),
        in_specs=[
            pl.BlockSpec((R, H), lambda i: (i, 0)),
            pl.BlockSpec((R, H), lambda i: (i, 0)),
            pl.BlockSpec((R, D), lambda i: (i, 0)),
            pl.BlockSpec((D, 3 * D), lambda i: (0, 0)),
            pl.BlockSpec((1, 3 * D), lambda i: (0, 0)),
            pl.BlockSpec((R, 3 * D), lambda i: (i, 0)),
        ],
        out_specs=pl.BlockSpec((R, D), lambda i: (i, 0)),
        out_shape=jax.ShapeDtypeStruct((N, D), jnp.float32),
    )(alo, ahi, h, wihT, bih, gh)


def _lin2_body(a_ref, b_ref, wta_ref, wtb_ref, bias_ref, out_ref):
    out = jnp.dot(a_ref[...], wta_ref[...], preferred_element_type=jnp.float32)
    out = out + jnp.dot(b_ref[...], wtb_ref[...],
                        preferred_element_type=jnp.float32)
    out_ref[...] = out + bias_ref[...]


def _lin2_call(a, b, wta, wtb, bias, out_dim):
    nb = N // R
    return pl.pallas_call(
        _lin2_body,
        grid=(nb,),
        in_specs=[
            pl.BlockSpec((R, D), lambda i: (i, 0)),
            pl.BlockSpec((R, D), lambda i: (i, 0)),
            pl.BlockSpec((D, out_dim), lambda i: (0, 0)),
            pl.BlockSpec((D, out_dim), lambda i: (0, 0)),
            pl.BlockSpec((1, out_dim), lambda i: (0, 0)),
        ],
        out_specs=pl.BlockSpec((R, out_dim), lambda i: (i, 0)),
        out_shape=jax.ShapeDtypeStruct((N, out_dim), jnp.float32),
    )(a, b, wta, wtb, bias)


def _make_scatter(KN):
    mesh = plsc.VectorSubcoreMesh(core_axis_name="c", subcore_axis_name="s")
    ept = E // NS
    nbatch = ept // EB
    rpt = N_PAD // NS

    @functools.partial(
        pl.kernel,
        out_type=[
            jax.ShapeDtypeStruct((N_PAD, H), jnp.float32),
            jax.ShapeDtypeStruct((N_PAD, H), jnp.float32),
        ],
        mesh=mesh,
        scratch_types=[
            pltpu.VMEM((ept,), jnp.int32),
            pltpu.VMEM((nbatch, EB), jnp.int32),
            pltpu.VMEM((EB, H), jnp.float32),
            pltpu.VMEM((EB, H), jnp.float32),
            pltpu.VMEM_SHARED((N_PAD, H), jnp.float32),
            pltpu.SemaphoreType.DMA,
            pltpu.SemaphoreType.DMA,
            pltpu.SemaphoreType.DMA,
            pltpu.SemaphoreType.DMA,
            pltpu.SemaphoreType.DMA,
        ],
    )
    def scatter_k(wh_lo, wh_hi, gidx_hbm, dst_hbm, zeros_hbm, out_lo, out_hi,
                  gidx_v, dst_v, rows0, rows1, acc, gsem0, gsem1, ssem0,
                  ssem1, isem):
        c = lax.axis_index("c")
        s = lax.axis_index("s")
        icp0 = pltpu.async_copy(gidx_hbm.at[s], gidx_v, isem)
        icp1 = pltpu.async_copy(dst_hbm.at[s], dst_v, isem)
        pltpu.sync_copy(zeros_hbm.at[pl.ds(s * rpt, rpt)],
                        acc.at[pl.ds(s * rpt, rpt)])
        icp0.wait()
        icp1.wait()
        plsc.subcore_barrier()

        rows = (rows0, rows1)
        gsem = (gsem0, gsem1)
        ssem = (ssem0, ssem1)

        def run(wh):
            def gather_start(i, b):
                pltpu.async_copy(wh.at[gidx_v.at[pl.ds(i * EB, EB)]],
                                 rows[b], gsem[b])

            def gather_wait(i, b):
                pltpu.make_async_copy(wh.at[gidx_v.at[pl.ds(i * EB, EB)]],
                                      rows[b], gsem[b]).wait()

            def scat_start(i, b):
                pltpu.async_copy(rows[b], acc.at[dst_v.at[i]], ssem[b],
                                 add=True)

            def scat_wait(i, b):
                pltpu.make_async_copy(rows[b], acc.at[dst_v.at[i]],
                                      ssem[b]).wait()

            gather_start(0, 0)

            @pl.loop(0, nbatch - 1, step=2)
            def _(g):
                gather_wait(g, 0)

                @pl.when(g > 0)
                def _():
                    scat_wait(g - 1, 1)

                gather_start(g + 1, 1)
                scat_start(g, 0)

                gather_wait(g + 1, 1)
                scat_wait(g, 0)
                gather_start(g + 2, 0)
                scat_start(g + 1, 1)

            gather_wait(nbatch - 1, 0)
            scat_wait(nbatch - 2, 1)
            pltpu.sync_copy(rows[0], acc.at[dst_v.at[nbatch - 1]], add=True)

        @pl.when(c == 0)
        def _():
            run(wh_lo)

        @pl.when(c == 1)
        def _():
            run(wh_hi)

        plsc.subcore_barrier()

        ob = s * rpt

        @pl.when(c == 0)
        def _():
            pltpu.sync_copy(acc.at[pl.ds(ob, rpt)], out_lo.at[pl.ds(ob, rpt)])

        @pl.when(c == 1)
        def _():
            pltpu.sync_copy(acc.at[pl.ds(ob, rpt)], out_hi.at[pl.ds(ob, rpt)])

    return scatter_k


_scatter_ast = _make_scatter(N)
_scatter_cpg = _make_scatter(3 * N)


def _conv(h, gidx, dst, zeros, wT, b, wihT, bih, whhT, bhh, scatter, K):
    nb = (E // NS) // EB
    gidx = gidx.reshape(NS, E // NS)
    dst = dst.reshape(NS, nb, EB)
    for _ in range(N_STEPS):
        lo, hi = _wh_call(h, wT, b, K)
        alo, ahi = scatter(lo, hi, gidx, dst, zeros)
        gh = _gh_call(h, whhT, bhh)
        h = _gru_call(alo[:N], ahi[:N], h, wihT, bih, gh)
    return h


def kernel(x, ast_edge_index, cpg_edge_index, cpg_etypes, ast_W, ast_b,
           ast_Wih, ast_bih, ast_Whh, ast_bhh, cpg_Ws, cpg_bs, cpg_Wih,
           cpg_bih, cpg_Whh, cpg_bhh, fn1_W, fn1_b, fn2_W, fn2_b):
    zeros = jnp.zeros((N_PAD, H), jnp.float32)

    gidx_a = ast_edge_index[0].astype(jnp.int32)
    dst_a = ast_edge_index[1].astype(jnp.int32)
    h_ast = _conv(
        x, gidx_a, dst_a, zeros,
        jnp.transpose(ast_W)[None], ast_b[None, None],
        jnp.transpose(ast_Wih), ast_bih[None], jnp.transpose(ast_Whh),
        ast_bhh[None], _scatter_ast, 1)

    fn1_WT = jnp.transpose(fn1_W)
    hiddens = _lin2_call(h_ast, x, fn1_WT[:D], fn1_WT[D:], fn1_b[None], D)

    gidx_c = (cpg_etypes.astype(jnp.int32) * N
              + cpg_edge_index[0].astype(jnp.int32))
    dst_c = cpg_edge_index[1].astype(jnp.int32)
    h_cpg = _conv(
        hiddens, gidx_c, dst_c, zeros,
        jnp.transpose(cpg_Ws, (0, 2, 1)), cpg_bs[:, None],
        jnp.transpose(cpg_Wih), cpg_bih[None], jnp.transpose(cpg_Whh),
        cpg_bhh[None], _scatter_cpg, 3)

    fn2_WT = jnp.transpose(fn2_W)
    logits = _lin2_call(h_cpg, hiddens, fn2_WT[:D], fn2_WT[D:], fn2_b[None], D)
    return logits


# speedup vs baseline: 1.0458x; 1.0458x over previous
import functools

import jax
import jax.numpy as jnp
from jax import lax
from jax.experimental import pallas as pl
from jax.experimental.pallas import tpu as pltpu
from jax.experimental.pallas import tpu_sc as plsc

N = 10000
N_PAD = 10240
D = 256
H = 128
E = 160000
N_STEPS = 2
NS = 16
EB = 80
R = 1000

_BF = jnp.bfloat16


def _wh_body(h_ref, wt_ref, b_ref, lo_ref, hi_ref):
    wh = jnp.dot(h_ref[...].astype(_BF), wt_ref[0],
                 preferred_element_type=jnp.float32)
    wh = wh + b_ref[0]
    lo_ref[...] = wh[:, :H]
    hi_ref[...] = wh[:, H:]


def _wh_call(h, wT, b, K):
    nb = N // R
    return pl.pallas_call(
        _wh_body,
        grid=(K, nb),
        in_specs=[
            pl.BlockSpec((R, D), lambda k, i: (i, 0)),
            pl.BlockSpec((1, D, D), lambda k, i: (k, 0, 0)),
            pl.BlockSpec((1, 1, D), lambda k, i: (k, 0, 0)),
        ],
        out_specs=[
            pl.BlockSpec((R, H), lambda k, i: (k * nb + i, 0)),
            pl.BlockSpec((R, H), lambda k, i: (k * nb + i, 0)),
        ],
        out_shape=[
            jax.ShapeDtypeStruct((K * N, H), jnp.float32),
            jax.ShapeDtypeStruct((K * N, H), jnp.float32),
        ],
    )(h, wT, b)


def _gru_body(alo_ref, ahi_ref, h_ref, wihT_ref, bih_ref, whhT_ref, bhh_ref,
              out_ref):
    gi = jnp.dot(alo_ref[...].astype(_BF), wihT_ref[:H],
                 preferred_element_type=jnp.float32)
    gi = gi + jnp.dot(ahi_ref[...].astype(_BF), wihT_ref[H:],
                      preferred_element_type=jnp.float32)
    gi = gi + bih_ref[...]
    h = h_ref[...]
    gh = jnp.dot(h.astype(_BF), whhT_ref[...],
                 preferred_element_type=jnp.float32)
    gh = gh + bhh_ref[...]
    r = jax.nn.sigmoid(gi[:, :D] + gh[:, :D])
    z = jax.nn.sigmoid(gi[:, D:2 * D] + gh[:, D:2 * D])
    n = jnp.tanh(gi[:, 2 * D:] + r * gh[:, 2 * D:])
    out_ref[...] = (1.0 - z) * n + z * h


def _gru_call(alo, ahi, h, wihT, bih, whhT, bhh):
    nb = N // R
    return pl.pallas_call(
        _gru_body,
        grid=(nb,),
        in_specs=[
            pl.BlockSpec((R, H), lambda i: (i, 0)),
            pl.BlockSpec((R, H), lambda i: (i, 0)),
            pl.BlockSpec((R, D), lambda i: (i, 0)),
            pl.BlockSpec((D, 3 * D), lambda i: (0, 0)),
            pl.BlockSpec((1, 3 * D), lambda i: (0, 0)),
            pl.BlockSpec((D, 3 * D), lambda i: (0, 0)),
            pl.BlockSpec((1, 3 * D), lambda i: (0, 0)),
        ],
        out_specs=pl.BlockSpec((R, D), lambda i: (i, 0)),
        out_shape=jax.ShapeDtypeStruct((N, D), jnp.float32),
    )(alo, ahi, h, wihT, bih, whhT, bhh)


def _lin2_body(a_ref, b_ref, wta_ref, wtb_ref, bias_ref, out_ref):
    out = jnp.dot(a_ref[...].astype(_BF), wta_ref[...],
                  preferred_element_type=jnp.float32)
    out = out + jnp.dot(b_ref[...].astype(_BF), wtb_ref[...],
                        preferred_element_type=jnp.float32)
    out_ref[...] = out + bias_ref[...]


def _lin2_call(a, b, wta, wtb, bias, out_dim):
    nb = N // R
    return pl.pallas_call(
        _lin2_body,
        grid=(nb,),
        in_specs=[
            pl.BlockSpec((R, D), lambda i: (i, 0)),
            pl.BlockSpec((R, D), lambda i: (i, 0)),
            pl.BlockSpec((D, out_dim), lambda i: (0, 0)),
            pl.BlockSpec((D, out_dim), lambda i: (0, 0)),
            pl.BlockSpec((1, out_dim), lambda i: (0, 0)),
        ],
        out_specs=pl.BlockSpec((R, out_dim), lambda i: (i, 0)),
        out_shape=jax.ShapeDtypeStruct((N, out_dim), jnp.float32),
    )(a, b, wta, wtb, bias)


def _make_scatter(KN):
    mesh = plsc.VectorSubcoreMesh(core_axis_name="c", subcore_axis_name="s")
    ept = E // NS
    nbatch = ept // EB
    rpt = N_PAD // NS

    @functools.partial(
        pl.kernel,
        out_type=[
            jax.ShapeDtypeStruct((N_PAD, H), jnp.float32),
            jax.ShapeDtypeStruct((N_PAD, H), jnp.float32),
        ],
        mesh=mesh,
        scratch_types=[
            pltpu.VMEM((ept,), jnp.int32),
            pltpu.VMEM((nbatch, EB), jnp.int32),
            pltpu.VMEM((EB, H), jnp.float32),
            pltpu.VMEM((EB, H), jnp.float32),
            pltpu.VMEM_SHARED((N_PAD, H), jnp.float32),
            pltpu.SemaphoreType.DMA,
            pltpu.SemaphoreType.DMA,
            pltpu.SemaphoreType.DMA,
            pltpu.SemaphoreType.DMA,
            pltpu.SemaphoreType.DMA,
        ],
    )
    def scatter_k(wh_lo, wh_hi, gidx_hbm, dst_hbm, zeros_hbm, out_lo, out_hi,
                  gidx_v, dst_v, rows0, rows1, acc, gsem0, gsem1, ssem0,
                  ssem1, isem):
        c = lax.axis_index("c")
        s = lax.axis_index("s")
        icp0 = pltpu.async_copy(gidx_hbm.at[s], gidx_v, isem)
        icp1 = pltpu.async_copy(dst_hbm.at[s], dst_v, isem)
        pltpu.sync_copy(zeros_hbm.at[pl.ds(s * rpt, rpt)],
                        acc.at[pl.ds(s * rpt, rpt)])
        icp0.wait()
        icp1.wait()
        plsc.subcore_barrier()

        rows = (rows0, rows1)
        gsem = (gsem0, gsem1)
        ssem = (ssem0, ssem1)

        def run(wh):
            def gather_start(i, b):
                pltpu.async_copy(wh.at[gidx_v.at[pl.ds(i * EB, EB)]],
                                 rows[b], gsem[b])

            def gather_wait(i, b):
                pltpu.make_async_copy(wh.at[gidx_v.at[pl.ds(i * EB, EB)]],
                                      rows[b], gsem[b]).wait()

            def scat_start(i, b):
                pltpu.async_copy(rows[b], acc.at[dst_v.at[i]], ssem[b],
                                 add=True)

            def scat_wait(i, b):
                pltpu.make_async_copy(rows[b], acc.at[dst_v.at[i]],
                                      ssem[b]).wait()

            gather_start(0, 0)

            @pl.loop(0, nbatch - 1, step=2)
            def _(g):
                gather_wait(g, 0)

                @pl.when(g > 0)
                def _():
                    scat_wait(g - 1, 1)

                gather_start(g + 1, 1)
                scat_start(g, 0)

                gather_wait(g + 1, 1)
                scat_wait(g, 0)
                gather_start(g + 2, 0)
                scat_start(g + 1, 1)

            gather_wait(nbatch - 1, 0)
            scat_wait(nbatch - 2, 1)
            pltpu.sync_copy(rows[0], acc.at[dst_v.at[nbatch - 1]], add=True)

        @pl.when(c == 0)
        def _():
            run(wh_lo)

        @pl.when(c == 1)
        def _():
            run(wh_hi)

        plsc.subcore_barrier()

        ob = s * rpt

        @pl.when(c == 0)
        def _():
            pltpu.sync_copy(acc.at[pl.ds(ob, rpt)], out_lo.at[pl.ds(ob, rpt)])

        @pl.when(c == 1)
        def _():
            pltpu.sync_copy(acc.at[pl.ds(ob, rpt)], out_hi.at[pl.ds(ob, rpt)])

    return scatter_k


_scatter_ast = _make_scatter(N)
_scatter_cpg = _make_scatter(3 * N)


def _conv(h, gidx, dst, zeros, wT, b, wihT, bih, whhT, bhh, scatter, K):
    nb = (E // NS) // EB
    gidx = gidx.reshape(NS, E // NS)
    dst = dst.reshape(NS, nb, EB)
    for _ in range(N_STEPS):
        lo, hi = _wh_call(h, wT, b, K)
        alo, ahi = scatter(lo, hi, gidx, dst, zeros)
        h = _gru_call(alo[:N], ahi[:N], h, wihT, bih, whhT, bhh)
    return h


def kernel(x, ast_edge_index, cpg_edge_index, cpg_etypes, ast_W, ast_b,
           ast_Wih, ast_bih, ast_Whh, ast_bhh, cpg_Ws, cpg_bs, cpg_Wih,
           cpg_bih, cpg_Whh, cpg_bhh, fn1_W, fn1_b, fn2_W, fn2_b):
    zeros = jnp.zeros((N_PAD, H), jnp.float32)

    gidx_a = ast_edge_index[0].astype(jnp.int32)
    dst_a = ast_edge_index[1].astype(jnp.int32)
    h_ast = _conv(
        x, gidx_a, dst_a, zeros,
        jnp.transpose(ast_W)[None].astype(_BF), ast_b[None, None],
        jnp.transpose(ast_Wih).astype(_BF), ast_bih[None],
        jnp.transpose(ast_Whh).astype(_BF), ast_bhh[None], _scatter_ast, 1)

    fn1_WT = jnp.transpose(fn1_W).astype(_BF)
    hiddens = _lin2_call(h_ast, x, fn1_WT[:D], fn1_WT[D:], fn1_b[None], D)

    gidx_c = (cpg_etypes.astype(jnp.int32) * N
              + cpg_edge_index[0].astype(jnp.int32))
    dst_c = cpg_edge_index[1].astype(jnp.int32)
    h_cpg = _conv(
        hiddens, gidx_c, dst_c, zeros,
        jnp.transpose(cpg_Ws, (0, 2, 1)).astype(_BF), cpg_bs[:, None],
        jnp.transpose(cpg_Wih).astype(_BF), cpg_bih[None],
        jnp.transpose(cpg_Whh).astype(_BF), cpg_bhh[None], _scatter_cpg, 3)

    fn2_WT = jnp.transpose(fn2_W).astype(_BF)
    logits = _lin2_call(h_cpg, hiddens, fn2_WT[:D], fn2_WT[D:], fn2_b[None], D)
    return logits


# speedup vs baseline: 1.1967x; 1.1444x over previous
import functools

import jax
import jax.numpy as jnp
from jax import lax
from jax.experimental import pallas as pl
from jax.experimental.pallas import tpu as pltpu
from jax.experimental.pallas import tpu_sc as plsc

N = 10000
N_PAD = 10240
D = 256
H = 128
E = 160000
N_STEPS = 2
NS = 16
EB = 80
R = 1000

_BF = jnp.bfloat16


def _wh_body(h_ref, wt_ref, b_ref, lo_ref, hi_ref):
    wh = jnp.dot(h_ref[...].astype(_BF), wt_ref[0],
                 preferred_element_type=jnp.float32)
    wh = wh + b_ref[0]
    lo_ref[...] = wh[:, :H]
    hi_ref[...] = wh[:, H:]


def _wh_call(h, wT, b, K):
    nb = N // R
    return pl.pallas_call(
        _wh_body,
        grid=(K, nb),
        in_specs=[
            pl.BlockSpec((R, D), lambda k, i: (i, 0)),
            pl.BlockSpec((1, D, D), lambda k, i: (k, 0, 0)),
            pl.BlockSpec((1, 1, D), lambda k, i: (k, 0, 0)),
        ],
        out_specs=[
            pl.BlockSpec((R, H), lambda k, i: (k * nb + i, 0)),
            pl.BlockSpec((R, H), lambda k, i: (k * nb + i, 0)),
        ],
        out_shape=[
            jax.ShapeDtypeStruct((K * N, H), jnp.float32),
            jax.ShapeDtypeStruct((K * N, H), jnp.float32),
        ],
    )(h, wT, b)


def _make_fused(K, with_lin):
    def body(*refs):
        (alo_ref, ahi_ref, h_ref, wihT_ref, bih_ref, whhT_ref,
         bhh_ref) = refs[:7]
        idx = 7
        if with_lin:
            x2_ref, wta_ref, wtb_ref, bias_ref = refs[idx:idx + 4]
            idx += 4
        if K:
            wt_ref, b_ref = refs[idx:idx + 2]
            idx += 2
        outs = refs[idx:]
        gi = jnp.dot(alo_ref[...].astype(_BF), wihT_ref[:H],
                     preferred_element_type=jnp.float32)
        gi = gi + jnp.dot(ahi_ref[...].astype(_BF), wihT_ref[H:],
                          preferred_element_type=jnp.float32)
        gi = gi + bih_ref[...]
        h = h_ref[...]
        gh = jnp.dot(h.astype(_BF), whhT_ref[...],
                     preferred_element_type=jnp.float32)
        gh = gh + bhh_ref[...]
        r = jax.nn.sigmoid(gi[:, :D] + gh[:, :D])
        z = jax.nn.sigmoid(gi[:, D:2 * D] + gh[:, D:2 * D])
        n = jnp.tanh(gi[:, 2 * D:] + r * gh[:, 2 * D:])
        h_new = (1.0 - z) * n + z * h
        if with_lin:
            main = jnp.dot(h_new.astype(_BF), wta_ref[...],
                           preferred_element_type=jnp.float32)
            main = main + jnp.dot(x2_ref[...].astype(_BF), wtb_ref[...],
                                  preferred_element_type=jnp.float32)
            main = main + bias_ref[...]
        else:
            main = h_new
        outs[0][...] = main
        if K:
            mb = main.astype(_BF)
            for k in range(K):
                whk = jnp.dot(mb, wt_ref[k], preferred_element_type=jnp.float32)
                whk = whk + b_ref[k]
                outs[1][k] = whk[:, :H]
                outs[2][k] = whk[:, H:]

    nb = N // R
    in_specs = [
        pl.BlockSpec((R, H), lambda i: (i, 0)),
        pl.BlockSpec((R, H), lambda i: (i, 0)),
        pl.BlockSpec((R, D), lambda i: (i, 0)),
        pl.BlockSpec((D, 3 * D), lambda i: (0, 0)),
        pl.BlockSpec((1, 3 * D), lambda i: (0, 0)),
        pl.BlockSpec((D, 3 * D), lambda i: (0, 0)),
        pl.BlockSpec((1, 3 * D), lambda i: (0, 0)),
    ]
    if with_lin:
        in_specs += [
            pl.BlockSpec((R, D), lambda i: (i, 0)),
            pl.BlockSpec((D, D), lambda i: (0, 0)),
            pl.BlockSpec((D, D), lambda i: (0, 0)),
            pl.BlockSpec((1, D), lambda i: (0, 0)),
        ]
    if K:
        in_specs += [
            pl.BlockSpec((K, D, D), lambda i: (0, 0, 0)),
            pl.BlockSpec((K, 1, D), lambda i: (0, 0, 0)),
        ]
    out_specs = [pl.BlockSpec((R, D), lambda i: (i, 0))]
    out_shape = [jax.ShapeDtypeStruct((N, D), jnp.float32)]
    if K:
        out_specs += [pl.BlockSpec((K, R, H), lambda i: (0, i, 0))] * 2
        out_shape += [jax.ShapeDtypeStruct((K, N, H), jnp.float32)] * 2

    call = pl.pallas_call(body, grid=(nb,), in_specs=in_specs,
                          out_specs=out_specs, out_shape=out_shape)

    def run(*args):
        res = call(*args)
        if K:
            main, lo, hi = res
            return main, lo.reshape(K * N, H), hi.reshape(K * N, H)
        return res[0]

    return run


_fused_ast_wh1 = _make_fused(1, False)
_fused_fn1_cpg = _make_fused(3, True)
_fused_cpg_wh1 = _make_fused(3, False)
_fused_fn2 = _make_fused(0, True)


def _make_scatter(KN):
    mesh = plsc.VectorSubcoreMesh(core_axis_name="c", subcore_axis_name="s")
    ept = E // NS
    nbatch = ept // EB
    rpt = N_PAD // NS

    @functools.partial(
        pl.kernel,
        out_type=[
            jax.ShapeDtypeStruct((N_PAD, H), jnp.float32),
            jax.ShapeDtypeStruct((N_PAD, H), jnp.float32),
        ],
        mesh=mesh,
        scratch_types=[
            pltpu.VMEM((ept,), jnp.int32),
            pltpu.VMEM((nbatch, EB), jnp.int32),
            pltpu.VMEM((EB, H), jnp.float32),
            pltpu.VMEM((EB, H), jnp.float32),
            pltpu.VMEM_SHARED((N_PAD, H), jnp.float32),
            pltpu.SemaphoreType.DMA,
            pltpu.SemaphoreType.DMA,
            pltpu.SemaphoreType.DMA,
            pltpu.SemaphoreType.DMA,
            pltpu.SemaphoreType.DMA,
        ],
    )
    def scatter_k(wh_lo, wh_hi, gidx_hbm, dst_hbm, zeros_hbm, out_lo, out_hi,
                  gidx_v, dst_v, rows0, rows1, acc, gsem0, gsem1, ssem0,
                  ssem1, isem):
        c = lax.axis_index("c")
        s = lax.axis_index("s")
        icp0 = pltpu.async_copy(gidx_hbm.at[s], gidx_v, isem)
        icp1 = pltpu.async_copy(dst_hbm.at[s], dst_v, isem)
        pltpu.sync_copy(zeros_hbm.at[pl.ds(s * rpt, rpt)],
                        acc.at[pl.ds(s * rpt, rpt)])
        icp0.wait()
        icp1.wait()
        plsc.subcore_barrier()

        rows = (rows0, rows1)
        gsem = (gsem0, gsem1)
        ssem = (ssem0, ssem1)

        def run(wh):
            def gather_start(i, b):
                pltpu.async_copy(wh.at[gidx_v.at[pl.ds(i * EB, EB)]],
                                 rows[b], gsem[b])

            def gather_wait(i, b):
                pltpu.make_async_copy(wh.at[gidx_v.at[pl.ds(i * EB, EB)]],
                                      rows[b], gsem[b]).wait()

            def scat_start(i, b):
                pltpu.async_copy(rows[b], acc.at[dst_v.at[i]], ssem[b],
                                 add=True)

            def scat_wait(i, b):
                pltpu.make_async_copy(rows[b], acc.at[dst_v.at[i]],
                                      ssem[b]).wait()

            gather_start(0, 0)

            @pl.loop(0, nbatch - 1, step=2)
            def _(g):
                gather_wait(g, 0)

                @pl.when(g > 0)
                def _():
                    scat_wait(g - 1, 1)

                gather_start(g + 1, 1)
                scat_start(g, 0)

                gather_wait(g + 1, 1)
                scat_wait(g, 0)
                gather_start(g + 2, 0)
                scat_start(g + 1, 1)

            gather_wait(nbatch - 1, 0)
            scat_wait(nbatch - 2, 1)
            pltpu.sync_copy(rows[0], acc.at[dst_v.at[nbatch - 1]], add=True)

        @pl.when(c == 0)
        def _():
            run(wh_lo)

        @pl.when(c == 1)
        def _():
            run(wh_hi)

        plsc.subcore_barrier()

        ob = s * rpt

        @pl.when(c == 0)
        def _():
            pltpu.sync_copy(acc.at[pl.ds(ob, rpt)], out_lo.at[pl.ds(ob, rpt)])

        @pl.when(c == 1)
        def _():
            pltpu.sync_copy(acc.at[pl.ds(ob, rpt)], out_hi.at[pl.ds(ob, rpt)])

    return scatter_k


_scatter_ast = _make_scatter(N)
_scatter_cpg = _make_scatter(3 * N)


def kernel(x, ast_edge_index, cpg_edge_index, cpg_etypes, ast_W, ast_b,
           ast_Wih, ast_bih, ast_Whh, ast_bhh, cpg_Ws, cpg_bs, cpg_Wih,
           cpg_bih, cpg_Whh, cpg_bhh, fn1_W, fn1_b, fn2_W, fn2_b):
    zeros = jnp.zeros((N_PAD, H), jnp.float32)
    nbt = (E // NS) // EB

    gidx_a = ast_edge_index[0].astype(jnp.int32).reshape(NS, E // NS)
    dst_a = ast_edge_index[1].astype(jnp.int32).reshape(NS, nbt, EB)
    gidx_c = (cpg_etypes.astype(jnp.int32) * N
              + cpg_edge_index[0].astype(jnp.int32)).reshape(NS, E // NS)
    dst_c = cpg_edge_index[1].astype(jnp.int32).reshape(NS, nbt, EB)

    astWT = jnp.transpose(ast_W)[None].astype(_BF)
    astB = ast_b[None, None]
    ast_gru = (jnp.transpose(ast_Wih).astype(_BF), ast_bih[None],
               jnp.transpose(ast_Whh).astype(_BF), ast_bhh[None])
    cpgWT = jnp.transpose(cpg_Ws, (0, 2, 1)).astype(_BF)
    cpgB = cpg_bs[:, None]
    cpg_gru = (jnp.transpose(cpg_Wih).astype(_BF), cpg_bih[None],
               jnp.transpose(cpg_Whh).astype(_BF), cpg_bhh[None])
    fn1_WT = jnp.transpose(fn1_W).astype(_BF)
    fn2_WT = jnp.transpose(fn2_W).astype(_BF)

    lo, hi = _wh_call(x, astWT, astB, 1)
    alo, ahi = _scatter_ast(lo, hi, gidx_a, dst_a, zeros)
    h1, lo, hi = _fused_ast_wh1(alo, ahi, x, *ast_gru, astWT, astB)
    alo, ahi = _scatter_ast(lo, hi, gidx_a, dst_a, zeros)
    hiddens, lo, hi = _fused_fn1_cpg(alo, ahi, h1, *ast_gru, x,
                                     fn1_WT[:D], fn1_WT[D:], fn1_b[None],
                                     cpgWT, cpgB)
    alo, ahi = _scatter_cpg(lo, hi, gidx_c, dst_c, zeros)
    h1c, lo, hi = _fused_cpg_wh1(alo, ahi, hiddens, *cpg_gru, cpgWT, cpgB)
    alo, ahi = _scatter_cpg(lo, hi, gidx_c, dst_c, zeros)
    logits = _fused_fn2(alo, ahi, h1c, *cpg_gru, hiddens,
                        fn2_WT[:D], fn2_WT[D:], fn2_b[None])
    return logits
